# Initial kernel scaffold; baseline (speedup 1.0000x reference)
import jax
import jax.numpy as jnp
from jax.experimental import pallas as pl


def kernel(x, norm1_w, norm1_b, qkv_w, qkv_dw_w, temperature, proj_w, norm2_w, norm2_b, ffn_in_w, ffn_dw_w, ffn_out_w):
    raise NotImplementedError("write your pallas kernel here")


# speedup vs baseline: 87.0172x; 87.0172x over previous
import jax
import jax.numpy as jnp
from jax.experimental import pallas as pl
from jax.experimental.pallas import tpu as pltpu

DIM = 128
HEADS = 4
T = 2
HF = 64
H = W = 32
B = 2
POS = H * W
N = POS * T
CG = DIM // T
CPH = CG // HEADS
BH = B * HEADS
KTOP = int(N * 0.25)
ROWS = 512
ITERS = 22


def _ln(x, g, b):
    mu = jnp.mean(x, axis=1, keepdims=True)
    xc = x - mu
    var = jnp.mean(xc * xc, axis=1, keepdims=True)
    return xc * jax.lax.rsqrt(var + 1e-5) * g + b


def _ln_qkv_kernel(x_ref, g_ref, b_ref, wmix_ref, y_ref):
    xn = _ln(x_ref[...], g_ref[...], b_ref[...])
    y_ref[...] = jnp.dot(xn, wmix_ref[...], preferred_element_type=jnp.float32)


def _dwconv_kernel(xp_ref, w_ref, o_ref):
    acc = xp_ref[0:H, 0:W, :] * w_ref[0:1, :].reshape(1, 1, -1)
    for tap in range(1, 9):
        dy, dx = tap // 3, tap % 3
        acc = acc + xp_ref[dy:dy + H, dx:dx + W, :] * w_ref[tap:tap + 1, :].reshape(1, 1, -1)
    o_ref[...] = acc


def _norm_kernel(q_ref, k_ref, t_ref, qo_ref, ko_ref):
    q = q_ref[...]
    k = k_ref[...]
    qn = jnp.maximum(jnp.sqrt(jnp.sum(q * q, axis=1, keepdims=True)), 1e-12)
    kn = jnp.maximum(jnp.sqrt(jnp.sum(k * k, axis=1, keepdims=True)), 1e-12)
    qo_ref[...] = (q / qn) * t_ref[...]
    ko_ref[...] = k / kn


def _attn_kernel(q_ref, k_ref, v_ref, o_ref):
    s = jax.lax.dot_general(q_ref[...], k_ref[...], (((0,), (0,)), ((), ())),
                            preferred_element_type=jnp.float32)
    rowmax = jnp.max(s, axis=1, keepdims=True)
    lo = jnp.min(s, axis=1, keepdims=True)
    hi = rowmax
    for _ in range(ITERS):
        mid = 0.5 * (lo + hi)
        cnt = jnp.sum(jnp.where(s >= mid, 1.0, 0.0), axis=1, keepdims=True)
        ge = cnt >= KTOP
        lo = jnp.where(ge, mid, lo)
        hi = jnp.where(ge, hi, mid)
    e = jnp.where(s >= lo, jnp.exp(s - rowmax), 0.0)
    z = jnp.sum(e, axis=1, keepdims=True)
    p = e / z
    o_ref[...] = jax.lax.dot_general(p, v_ref[...], (((1,), (0,)), ((), ())),
                                     preferred_element_type=jnp.float32)


def _mid_kernel(a_ref, x_ref, pw_ref, g_ref, b_ref, fw_ref, x1_ref, h0_ref):
    y = jax.lax.dot_general(a_ref[...], pw_ref[...], (((1,), (1,)), ((), ())),
                            preferred_element_type=jnp.float32)
    x1 = x_ref[...] + y
    x1_ref[...] = x1
    xn = _ln(x1, g_ref[...], b_ref[...])
    h0_ref[...] = jax.lax.dot_general(xn, fw_ref[...], (((1,), (1,)), ((), ())),
                                      preferred_element_type=jnp.float32)


def _ffn_kernel(xp_ref, w_ref, ow_ref, x1_ref, o_ref):
    acc = xp_ref[0:H, 0:W, :] * w_ref[0:1, :].reshape(1, 1, -1)
    for tap in range(1, 9):
        dy, dx = tap // 3, tap % 3
        acc = acc + xp_ref[dy:dy + H, dx:dx + W, :] * w_ref[tap:tap + 1, :].reshape(1, 1, -1)
    hact = jnp.maximum(acc, 0.0).reshape(POS, HF)
    y = jax.lax.dot_general(hact, ow_ref[...], (((1,), (1,)), ((), ())),
                            preferred_element_type=jnp.float32)
    o_ref[...] = x1_ref[...] + y


def _full(shape):
    nd = len(shape)
    return pl.BlockSpec(shape, lambda *_: (0,) * nd)


def kernel(x, norm1_w, norm1_b, qkv_w, qkv_dw_w, temperature, proj_w,
           norm2_w, norm2_b, ffn_in_w, ffn_dw_w, ffn_out_w):
    f32 = jnp.float32

    x_t = x.transpose(0, 2, 3, 1).reshape(B * POS, DIM)

    wmix = jnp.einsum('ot,cd->tcod', qkv_w, jnp.eye(CG, dtype=f32)).reshape(DIM, 3 * T * CG)

    y = pl.pallas_call(
        _ln_qkv_kernel,
        out_shape=jax.ShapeDtypeStruct((B * POS, 3 * T * CG), f32),
        in_specs=[_full((B * POS, DIM)), _full((1, DIM)), _full((1, DIM)),
                  _full((DIM, 3 * T * CG))],
        out_specs=_full((B * POS, 3 * T * CG)),
    )(x_t, norm1_w.reshape(1, DIM), norm1_b.reshape(1, DIM), wmix)

    yimg = y.reshape(B, H, W, 3 * T * CG)
    ypad = jnp.pad(yimg, ((0, 0), (1, 1), (1, 1), (0, 0)))
    wtap_q = jnp.repeat(qkv_dw_w[:, 0].transpose(1, 2, 0).reshape(9, 3 * T), CG, axis=1)

    conv = pl.pallas_call(
        _dwconv_kernel,
        grid=(B,),
        out_shape=jax.ShapeDtypeStruct((B, H, W, 3 * T * CG), f32),
        in_specs=[pl.BlockSpec((None, H + 2, W + 2, 3 * T * CG), lambda i: (i, 0, 0, 0)),
                  pl.BlockSpec((9, 3 * T * CG), lambda i: (0, 0))],
        out_specs=pl.BlockSpec((None, H, W, 3 * T * CG), lambda i: (i, 0, 0, 0)),
    )(ypad, wtap_q)

    c5 = conv.reshape(B, POS, 3 * T, HEADS, CPH)
    q8 = c5[:, :, 0:T].transpose(0, 3, 4, 1, 2).reshape(BH, CPH, N)
    k8 = c5[:, :, T:2 * T].transpose(0, 3, 4, 1, 2).reshape(BH, CPH, N)
    v8 = c5[:, :, 2 * T:3 * T].transpose(0, 3, 1, 2, 4).reshape(BH, N, CPH)
    tvec = jnp.broadcast_to(temperature.reshape(1, HEADS), (B, HEADS)).reshape(BH, 1, 1)

    qn, kn = pl.pallas_call(
        _norm_kernel,
        grid=(BH,),
        out_shape=[jax.ShapeDtypeStruct((BH, CPH, N), f32),
                   jax.ShapeDtypeStruct((BH, CPH, N), f32)],
        in_specs=[pl.BlockSpec((None, CPH, N), lambda i: (i, 0, 0)),
                  pl.BlockSpec((None, CPH, N), lambda i: (i, 0, 0)),
                  pl.BlockSpec((None, 1, 1), lambda i: (i, 0, 0))],
        out_specs=[pl.BlockSpec((None, CPH, N), lambda i: (i, 0, 0)),
                   pl.BlockSpec((None, CPH, N), lambda i: (i, 0, 0))],
    )(q8, k8, tvec)

    outt = pl.pallas_call(
        _attn_kernel,
        grid=(BH, N // ROWS),
        out_shape=jax.ShapeDtypeStruct((BH, N, CPH), f32),
        in_specs=[pl.BlockSpec((None, CPH, ROWS), lambda i, j: (i, 0, j)),
                  pl.BlockSpec((None, CPH, N), lambda i, j: (i, 0, 0)),
                  pl.BlockSpec((None, N, CPH), lambda i, j: (i, 0, 0))],
        out_specs=pl.BlockSpec((None, ROWS, CPH), lambda i, j: (i, j, 0)),
        compiler_params=pltpu.CompilerParams(
            dimension_semantics=("arbitrary", "arbitrary")),
    )(qn, kn, v8)

    a_t = outt.reshape(B, HEADS, POS, T, CPH).transpose(0, 2, 3, 1, 4).reshape(B * POS, DIM)

    x1, h0 = pl.pallas_call(
        _mid_kernel,
        out_shape=[jax.ShapeDtypeStruct((B * POS, DIM), f32),
                   jax.ShapeDtypeStruct((B * POS, HF), f32)],
        in_specs=[_full((B * POS, DIM)), _full((B * POS, DIM)), _full((DIM, DIM)),
                  _full((1, DIM)), _full((1, DIM)), _full((HF, DIM))],
        out_specs=[_full((B * POS, DIM)), _full((B * POS, HF))],
    )(a_t, x_t, proj_w, norm2_w.reshape(1, DIM), norm2_b.reshape(1, DIM), ffn_in_w)

    h0pad = jnp.pad(h0.reshape(B, H, W, HF), ((0, 0), (1, 1), (1, 1), (0, 0)))
    wtap_f = ffn_dw_w[:, 0].transpose(1, 2, 0).reshape(9, HF)

    out = pl.pallas_call(
        _ffn_kernel,
        grid=(B,),
        out_shape=jax.ShapeDtypeStruct((B, POS, DIM), f32),
        in_specs=[pl.BlockSpec((None, H + 2, W + 2, HF), lambda i: (i, 0, 0, 0)),
                  pl.BlockSpec((9, HF), lambda i: (0, 0)),
                  pl.BlockSpec((DIM, HF), lambda i: (0, 0)),
                  pl.BlockSpec((None, POS, DIM), lambda i: (i, 0, 0))],
        out_specs=pl.BlockSpec((None, POS, DIM), lambda i: (i, 0, 0)),
    )(h0pad, wtap_f, ffn_out_w, x1.reshape(B, POS, DIM))

    return out.reshape(B, H, W, DIM).transpose(0, 3, 1, 2)


# speedup vs baseline: 110.9796x; 1.2754x over previous
import jax
import jax.numpy as jnp
from jax.experimental import pallas as pl
from jax.experimental.pallas import tpu as pltpu

DIM = 128
HEADS = 4
T = 2
HF = 64
H = W = 32
B = 2
POS = H * W
N = POS * T
CG = DIM // T
CPH = CG // HEADS
BH = B * HEADS
KTOP = int(N * 0.25)
ROWS = 512
ITERS = 14


def _ln(x, g, b):
    mu = jnp.mean(x, axis=1, keepdims=True)
    xc = x - mu
    var = jnp.mean(xc * xc, axis=1, keepdims=True)
    return xc * jax.lax.rsqrt(var + 1e-5) * g + b


def _ln_qkv_kernel(x_ref, g_ref, b_ref, wmix_ref, y_ref):
    xn = _ln(x_ref[...], g_ref[...], b_ref[...])
    y_ref[...] = jnp.dot(xn, wmix_ref[...], preferred_element_type=jnp.float32)


def _dwconv_kernel(xp_ref, w_ref, o_ref):
    acc = xp_ref[0:H, 0:W, :] * w_ref[0:1, :].reshape(1, 1, -1)
    for tap in range(1, 9):
        dy, dx = tap // 3, tap % 3
        acc = acc + xp_ref[dy:dy + H, dx:dx + W, :] * w_ref[tap:tap + 1, :].reshape(1, 1, -1)
    o_ref[...] = acc


def _norm_kernel(q_ref, k_ref, t_ref, qo_ref, ko_ref):
    q = q_ref[...]
    k = k_ref[...]
    qn = jnp.maximum(jnp.sqrt(jnp.sum(q * q, axis=1, keepdims=True)), 1e-12)
    kn = jnp.maximum(jnp.sqrt(jnp.sum(k * k, axis=1, keepdims=True)), 1e-12)
    qo_ref[...] = (q / qn) * t_ref[...]
    ko_ref[...] = k / kn


def _attn_kernel(q_ref, k_ref, v_ref, o_ref):
    s = jax.lax.dot_general(q_ref[...], k_ref[...], (((0,), (0,)), ((), ())),
                            preferred_element_type=jnp.float32)
    rowmax = jnp.max(s, axis=1, keepdims=True)
    lo = jnp.min(s, axis=1, keepdims=True)
    hi = rowmax
    for _ in range(ITERS):
        mid = 0.5 * (lo + hi)
        cnt = jnp.sum(jnp.where(s >= mid, 1.0, 0.0), axis=1, keepdims=True)
        ge = cnt >= KTOP
        lo = jnp.where(ge, mid, lo)
        hi = jnp.where(ge, hi, mid)
    e = jnp.where(s >= lo, jnp.exp(s - rowmax), 0.0)
    z = jnp.sum(e, axis=1, keepdims=True)
    p = e / z
    o_ref[...] = jax.lax.dot_general(p, v_ref[...], (((1,), (0,)), ((), ())),
                                     preferred_element_type=jnp.float32)


def _mid_kernel(a_ref, x_ref, pw_ref, g_ref, b_ref, fw_ref, x1_ref, h0_ref):
    y = jax.lax.dot_general(a_ref[...], pw_ref[...], (((1,), (1,)), ((), ())),
                            preferred_element_type=jnp.float32)
    x1 = x_ref[...] + y
    x1_ref[...] = x1
    xn = _ln(x1, g_ref[...], b_ref[...])
    h0_ref[...] = jax.lax.dot_general(xn, fw_ref[...], (((1,), (1,)), ((), ())),
                                      preferred_element_type=jnp.float32)


def _ffn_kernel(xp_ref, w_ref, ow_ref, x1_ref, o_ref):
    acc = xp_ref[0:H, 0:W, :] * w_ref[0:1, :].reshape(1, 1, -1)
    for tap in range(1, 9):
        dy, dx = tap // 3, tap % 3
        acc = acc + xp_ref[dy:dy + H, dx:dx + W, :] * w_ref[tap:tap + 1, :].reshape(1, 1, -1)
    hact = jnp.maximum(acc, 0.0).reshape(POS, HF)
    y = jax.lax.dot_general(hact, ow_ref[...], (((1,), (1,)), ((), ())),
                            preferred_element_type=jnp.float32)
    o_ref[...] = x1_ref[...] + y


def _full(shape):
    nd = len(shape)
    return pl.BlockSpec(shape, lambda *_: (0,) * nd)


def kernel(x, norm1_w, norm1_b, qkv_w, qkv_dw_w, temperature, proj_w,
           norm2_w, norm2_b, ffn_in_w, ffn_dw_w, ffn_out_w):
    f32 = jnp.float32

    x_t = x.transpose(0, 2, 3, 1).reshape(B * POS, DIM)

    wmix = jnp.einsum('ot,cd->tcod', qkv_w, jnp.eye(CG, dtype=f32)).reshape(DIM, 3 * T * CG)

    y = pl.pallas_call(
        _ln_qkv_kernel,
        out_shape=jax.ShapeDtypeStruct((B * POS, 3 * T * CG), f32),
        in_specs=[_full((B * POS, DIM)), _full((1, DIM)), _full((1, DIM)),
                  _full((DIM, 3 * T * CG))],
        out_specs=_full((B * POS, 3 * T * CG)),
    )(x_t, norm1_w.reshape(1, DIM), norm1_b.reshape(1, DIM), wmix)

    yimg = y.reshape(B, H, W, 3 * T * CG)
    ypad = jnp.pad(yimg, ((0, 0), (1, 1), (1, 1), (0, 0)))
    wtap_q = jnp.repeat(qkv_dw_w[:, 0].transpose(1, 2, 0).reshape(9, 3 * T), CG, axis=1)

    conv = pl.pallas_call(
        _dwconv_kernel,
        grid=(B,),
        out_shape=jax.ShapeDtypeStruct((B, H, W, 3 * T * CG), f32),
        in_specs=[pl.BlockSpec((None, H + 2, W + 2, 3 * T * CG), lambda i: (i, 0, 0, 0)),
                  pl.BlockSpec((9, 3 * T * CG), lambda i: (0, 0))],
        out_specs=pl.BlockSpec((None, H, W, 3 * T * CG), lambda i: (i, 0, 0, 0)),
    )(ypad, wtap_q)

    c5 = conv.reshape(B, POS, 3 * T, HEADS, CPH)
    q8 = c5[:, :, 0:T].transpose(0, 3, 4, 1, 2).reshape(BH, CPH, N)
    k8 = c5[:, :, T:2 * T].transpose(0, 3, 4, 1, 2).reshape(BH, CPH, N)
    v8 = c5[:, :, 2 * T:3 * T].transpose(0, 3, 1, 2, 4).reshape(BH, N, CPH)
    tvec = jnp.broadcast_to(temperature.reshape(1, HEADS), (B, HEADS)).reshape(BH, 1, 1)

    qn, kn = pl.pallas_call(
        _norm_kernel,
        grid=(BH,),
        out_shape=[jax.ShapeDtypeStruct((BH, CPH, N), f32),
                   jax.ShapeDtypeStruct((BH, CPH, N), f32)],
        in_specs=[pl.BlockSpec((None, CPH, N), lambda i: (i, 0, 0)),
                  pl.BlockSpec((None, CPH, N), lambda i: (i, 0, 0)),
                  pl.BlockSpec((None, 1, 1), lambda i: (i, 0, 0))],
        out_specs=[pl.BlockSpec((None, CPH, N), lambda i: (i, 0, 0)),
                   pl.BlockSpec((None, CPH, N), lambda i: (i, 0, 0))],
    )(q8, k8, tvec)

    outt = pl.pallas_call(
        _attn_kernel,
        grid=(BH, N // ROWS),
        out_shape=jax.ShapeDtypeStruct((BH, N, CPH), f32),
        in_specs=[pl.BlockSpec((None, CPH, ROWS), lambda i, j: (i, 0, j)),
                  pl.BlockSpec((None, CPH, N), lambda i, j: (i, 0, 0)),
                  pl.BlockSpec((None, N, CPH), lambda i, j: (i, 0, 0))],
        out_specs=pl.BlockSpec((None, ROWS, CPH), lambda i, j: (i, j, 0)),
        compiler_params=pltpu.CompilerParams(
            dimension_semantics=("arbitrary", "arbitrary")),
    )(qn, kn, v8)

    a_t = outt.reshape(B, HEADS, POS, T, CPH).transpose(0, 2, 3, 1, 4).reshape(B * POS, DIM)

    x1, h0 = pl.pallas_call(
        _mid_kernel,
        out_shape=[jax.ShapeDtypeStruct((B * POS, DIM), f32),
                   jax.ShapeDtypeStruct((B * POS, HF), f32)],
        in_specs=[_full((B * POS, DIM)), _full((B * POS, DIM)), _full((DIM, DIM)),
                  _full((1, DIM)), _full((1, DIM)), _full((HF, DIM))],
        out_specs=[_full((B * POS, DIM)), _full((B * POS, HF))],
    )(a_t, x_t, proj_w, norm2_w.reshape(1, DIM), norm2_b.reshape(1, DIM), ffn_in_w)

    h0pad = jnp.pad(h0.reshape(B, H, W, HF), ((0, 0), (1, 1), (1, 1), (0, 0)))
    wtap_f = ffn_dw_w[:, 0].transpose(1, 2, 0).reshape(9, HF)

    out = pl.pallas_call(
        _ffn_kernel,
        grid=(B,),
        out_shape=jax.ShapeDtypeStruct((B, POS, DIM), f32),
        in_specs=[pl.BlockSpec((None, H + 2, W + 2, HF), lambda i: (i, 0, 0, 0)),
                  pl.BlockSpec((9, HF), lambda i: (0, 0)),
                  pl.BlockSpec((DIM, HF), lambda i: (0, 0)),
                  pl.BlockSpec((None, POS, DIM), lambda i: (i, 0, 0))],
        out_specs=pl.BlockSpec((None, POS, DIM), lambda i: (i, 0, 0)),
    )(h0pad, wtap_f, ffn_out_w, x1.reshape(B, POS, DIM))

    return out.reshape(B, H, W, DIM).transpose(0, 3, 1, 2)


# speedup vs baseline: 133.2220x; 1.2004x over previous
import jax
import jax.numpy as jnp
from jax.experimental import pallas as pl
from jax.experimental.pallas import tpu as pltpu

DIM = 128
HEADS = 4
T = 2
HF = 64
H = W = 32
B = 2
POS = H * W
N = POS * T
CG = DIM // T
CPH = CG // HEADS
BH = B * HEADS
KTOP = int(N * 0.25)
ROWS = 512
ITERS = 14


def _ln_cm(x, g, b):
    mu = jnp.mean(x, axis=0, keepdims=True)
    xc = x - mu
    var = jnp.mean(xc * xc, axis=0, keepdims=True)
    return xc * jax.lax.rsqrt(var + 1e-5) * g + b


def _ln_tm(x, g, b):
    mu = jnp.mean(x, axis=1, keepdims=True)
    xc = x - mu
    var = jnp.mean(xc * xc, axis=1, keepdims=True)
    return xc * jax.lax.rsqrt(var + 1e-5) * g + b


def _ln_qkv_kernel(x_ref, g_ref, b_ref, wmix_ref, y_ref):
    xn = _ln_cm(x_ref[...], g_ref[...], b_ref[...])
    y_ref[...] = jax.lax.dot_general(xn, wmix_ref[...], (((0,), (0,)), ((), ())),
                                     preferred_element_type=jnp.float32)


def _conv9(xp_ref, w_ref):
    acc = xp_ref[0:H, 0:W, :] * w_ref[0:1, :].reshape(1, 1, -1)
    for tap in range(1, 9):
        dy, dx = tap // 3, tap % 3
        acc = acc + xp_ref[dy:dy + H, dx:dx + W, :] * w_ref[tap:tap + 1, :].reshape(1, 1, -1)
    return acc


def _dwconv_kernel(xp_ref, w_ref, t_ref, q_ref, k_ref, v_ref):
    acc = _conv9(xp_ref, w_ref).reshape(POS, 3 * T * CG)
    qs = [jnp.concatenate([acc[:, t * CG + h2 * CPH:t * CG + (h2 + 1) * CPH]
                           for t in range(T)], axis=0) for h2 in range(HEADS)]
    vs = [jnp.concatenate([acc[:, (2 * T + t) * CG + h2 * CPH:(2 * T + t) * CG + (h2 + 1) * CPH]
                           for t in range(T)], axis=0) for h2 in range(HEADS)]
    q = jnp.stack(qs)
    v = jnp.stack(vs)
    k = jnp.concatenate([acc[:, (T + t) * CG:(T + 1 + t) * CG].T
                         for t in range(T)], axis=1)
    qn = jnp.maximum(jnp.sqrt(jnp.sum(q * q, axis=1, keepdims=True)), 1e-12)
    kn = jnp.maximum(jnp.sqrt(jnp.sum(k * k, axis=1, keepdims=True)), 1e-12)
    q_ref[...] = (q / qn) * t_ref[...]
    k_ref[...] = k / kn
    v_ref[...] = v


def _attn_kernel(q_ref, k_ref, v_ref, o_ref):
    s = jax.lax.dot_general(q_ref[...], k_ref[...], (((1,), (0,)), ((), ())),
                            preferred_element_type=jnp.float32)
    rowmax = jnp.max(s, axis=1, keepdims=True)
    lo = jnp.min(s, axis=1, keepdims=True)
    hi = rowmax
    for _ in range(ITERS):
        mid = 0.5 * (lo + hi)
        cnt = jnp.sum(jnp.where(s >= mid, 1.0, 0.0), axis=1, keepdims=True)
        ge = cnt >= KTOP
        lo = jnp.where(ge, mid, lo)
        hi = jnp.where(ge, hi, mid)
    e = jnp.where(s >= lo, jnp.exp(s - rowmax), 0.0)
    z = jnp.sum(e, axis=1, keepdims=True)
    o_ref[...] = jax.lax.dot_general(e / z, v_ref[...], (((1,), (0,)), ((), ())),
                                     preferred_element_type=jnp.float32)


def _mid_kernel(a_ref, x_ref, pw_ref, g_ref, b_ref, fw_ref, x1_ref, h0_ref):
    a = jnp.concatenate([a_ref[h2, t] for t in range(T) for h2 in range(HEADS)],
                        axis=1)
    y = jax.lax.dot_general(a, pw_ref[...], (((1,), (1,)), ((), ())),
                            preferred_element_type=jnp.float32)
    x1 = x_ref[...].T + y
    x1_ref[...] = x1
    xn = _ln_tm(x1, g_ref[...], b_ref[...])
    h0_ref[...] = jax.lax.dot_general(xn, fw_ref[...], (((1,), (1,)), ((), ())),
                                      preferred_element_type=jnp.float32)


def _ffn_kernel(xp_ref, w_ref, ow_ref, x1_ref, o_ref):
    acc = _conv9(xp_ref, w_ref)
    hact = jnp.maximum(acc, 0.0).reshape(POS, HF)
    y = jax.lax.dot_general(hact, ow_ref[...], (((1,), (1,)), ((), ())),
                            preferred_element_type=jnp.float32)
    o_ref[...] = (x1_ref[...] + y).T


def kernel(x, norm1_w, norm1_b, qkv_w, qkv_dw_w, temperature, proj_w,
           norm2_w, norm2_b, ffn_in_w, ffn_dw_w, ffn_out_w):
    f32 = jnp.float32
    x_cm = x.reshape(B, DIM, POS)

    wmix = jnp.einsum('ot,cd->tcod', qkv_w, jnp.eye(CG, dtype=f32)).reshape(DIM, 3 * T * CG)

    y = pl.pallas_call(
        _ln_qkv_kernel,
        grid=(B,),
        out_shape=jax.ShapeDtypeStruct((B, POS, 3 * T * CG), f32),
        in_specs=[pl.BlockSpec((None, DIM, POS), lambda i: (i, 0, 0)),
                  pl.BlockSpec((DIM, 1), lambda i: (0, 0)),
                  pl.BlockSpec((DIM, 1), lambda i: (0, 0)),
                  pl.BlockSpec((DIM, 3 * T * CG), lambda i: (0, 0))],
        out_specs=pl.BlockSpec((None, POS, 3 * T * CG), lambda i: (i, 0, 0)),
    )(x_cm, norm1_w.reshape(DIM, 1), norm1_b.reshape(DIM, 1), wmix)

    ypad = jnp.pad(y.reshape(B, H, W, 3 * T * CG), ((0, 0), (1, 1), (1, 1), (0, 0)))
    wtap_q = jnp.repeat(qkv_dw_w[:, 0].transpose(1, 2, 0).reshape(9, 3 * T), CG, axis=1)
    tvec = temperature.reshape(HEADS, 1, 1)

    qn, kn, v8 = pl.pallas_call(
        _dwconv_kernel,
        grid=(B,),
        out_shape=[jax.ShapeDtypeStruct((B, HEADS, N, CPH), f32),
                   jax.ShapeDtypeStruct((B, CG, N), f32),
                   jax.ShapeDtypeStruct((B, HEADS, N, CPH), f32)],
        in_specs=[pl.BlockSpec((None, H + 2, W + 2, 3 * T * CG), lambda i: (i, 0, 0, 0)),
                  pl.BlockSpec((9, 3 * T * CG), lambda i: (0, 0)),
                  pl.BlockSpec((HEADS, 1, 1), lambda i: (0, 0, 0))],
        out_specs=[pl.BlockSpec((None, HEADS, N, CPH), lambda i: (i, 0, 0, 0)),
                   pl.BlockSpec((None, CG, N), lambda i: (i, 0, 0)),
                   pl.BlockSpec((None, HEADS, N, CPH), lambda i: (i, 0, 0, 0))],
    )(ypad, wtap_q, tvec)

    qn = qn.reshape(BH, N, CPH)
    kn = kn.reshape(BH, CPH, N)
    v8 = v8.reshape(BH, N, CPH)

    outt = pl.pallas_call(
        _attn_kernel,
        grid=(BH, N // ROWS),
        out_shape=jax.ShapeDtypeStruct((BH, N, CPH), f32),
        in_specs=[pl.BlockSpec((None, ROWS, CPH), lambda i, j: (i, j, 0)),
                  pl.BlockSpec((None, CPH, N), lambda i, j: (i, 0, 0)),
                  pl.BlockSpec((None, N, CPH), lambda i, j: (i, 0, 0))],
        out_specs=pl.BlockSpec((None, ROWS, CPH), lambda i, j: (i, j, 0)),
        compiler_params=pltpu.CompilerParams(
            dimension_semantics=("arbitrary", "arbitrary")),
    )(qn, kn, v8)

    a5 = outt.reshape(B, HEADS, T, POS, CPH)
    x1, h0 = pl.pallas_call(
        _mid_kernel,
        grid=(B,),
        out_shape=[jax.ShapeDtypeStruct((B, POS, DIM), f32),
                   jax.ShapeDtypeStruct((B, POS, HF), f32)],
        in_specs=[pl.BlockSpec((None, HEADS, T, POS, CPH), lambda i: (i, 0, 0, 0, 0)),
                  pl.BlockSpec((None, DIM, POS), lambda i: (i, 0, 0)),
                  pl.BlockSpec((DIM, DIM), lambda i: (0, 0)),
                  pl.BlockSpec((1, DIM), lambda i: (0, 0)),
                  pl.BlockSpec((1, DIM), lambda i: (0, 0)),
                  pl.BlockSpec((HF, DIM), lambda i: (0, 0))],
        out_specs=[pl.BlockSpec((None, POS, DIM), lambda i: (i, 0, 0)),
                   pl.BlockSpec((None, POS, HF), lambda i: (i, 0, 0))],
    )(a5, x_cm, proj_w, norm2_w.reshape(1, DIM), norm2_b.reshape(1, DIM), ffn_in_w)

    h0pad = jnp.pad(h0.reshape(B, H, W, HF), ((0, 0), (1, 1), (1, 1), (0, 0)))
    wtap_f = ffn_dw_w[:, 0].transpose(1, 2, 0).reshape(9, HF)

    out = pl.pallas_call(
        _ffn_kernel,
        grid=(B,),
        out_shape=jax.ShapeDtypeStruct((B, DIM, POS), f32),
        in_specs=[pl.BlockSpec((None, H + 2, W + 2, HF), lambda i: (i, 0, 0, 0)),
                  pl.BlockSpec((9, HF), lambda i: (0, 0)),
                  pl.BlockSpec((DIM, HF), lambda i: (0, 0)),
                  pl.BlockSpec((None, POS, DIM), lambda i: (i, 0, 0))],
        out_specs=pl.BlockSpec((None, DIM, POS), lambda i: (i, 0, 0)),
    )(h0pad, wtap_f, ffn_out_w, x1)

    return out.reshape(B, DIM, H, W)


# speedup vs baseline: 147.6018x; 1.1079x over previous
import jax
import jax.numpy as jnp
from jax.experimental import pallas as pl
from jax.experimental.pallas import tpu as pltpu

DIM = 128
HEADS = 4
T = 2
HF = 64
H = W = 32
B = 2
POS = H * W
N = POS * T
CG = DIM // T
CPH = CG // HEADS
BH = B * HEADS
KTOP = int(N * 0.25)
ROWS = 512
ITERS = 12


def _ln_cm(x, g, b):
    mu = jnp.mean(x, axis=0, keepdims=True)
    xc = x - mu
    var = jnp.mean(xc * xc, axis=0, keepdims=True)
    return xc * jax.lax.rsqrt(var + 1e-5) * g + b


def _ln_tm(x, g, b):
    mu = jnp.mean(x, axis=1, keepdims=True)
    xc = x - mu
    var = jnp.mean(xc * xc, axis=1, keepdims=True)
    return xc * jax.lax.rsqrt(var + 1e-5) * g + b


def _ln_qkv_kernel(x_ref, g_ref, b_ref, wmix_ref, y_ref):
    xn = _ln_cm(x_ref[...], g_ref[...], b_ref[...])
    y_ref[...] = jax.lax.dot_general(xn, wmix_ref[...], (((0,), (0,)), ((), ())),
                                     preferred_element_type=jnp.float32)


def _conv9(xp_ref, w_ref):
    acc = xp_ref[0:H, 0:W, :] * w_ref[0:1, :].reshape(1, 1, -1)
    for tap in range(1, 9):
        dy, dx = tap // 3, tap % 3
        acc = acc + xp_ref[dy:dy + H, dx:dx + W, :] * w_ref[tap:tap + 1, :].reshape(1, 1, -1)
    return acc


def _dwconv_kernel(xp_ref, w_ref, t_ref, q_ref, k_ref, v_ref):
    acc = _conv9(xp_ref, w_ref).reshape(POS, 3 * T * CG)
    qs = [jnp.concatenate([acc[:, t * CG + h2 * CPH:t * CG + (h2 + 1) * CPH]
                           for t in range(T)], axis=0) for h2 in range(HEADS)]
    vs = [jnp.concatenate([acc[:, (2 * T + t) * CG + h2 * CPH:(2 * T + t) * CG + (h2 + 1) * CPH]
                           for t in range(T)], axis=0) for h2 in range(HEADS)]
    q = jnp.stack(qs)
    v = jnp.stack(vs)
    k = jnp.concatenate([acc[:, (T + t) * CG:(T + 1 + t) * CG].T
                         for t in range(T)], axis=1)
    qn = jnp.maximum(jnp.sqrt(jnp.sum(q * q, axis=1, keepdims=True)), 1e-12)
    kn = jnp.maximum(jnp.sqrt(jnp.sum(k * k, axis=1, keepdims=True)), 1e-12)
    q_ref[...] = (q / qn) * t_ref[...]
    k_ref[...] = k / kn
    v_ref[...] = v


def _attn_kernel(q_ref, k_ref, v_ref, o_ref):
    s = jax.lax.dot_general(q_ref[...], k_ref[...], (((1,), (0,)), ((), ())),
                            preferred_element_type=jnp.float32)
    rowmax = jnp.max(s, axis=1, keepdims=True)
    lo = jnp.min(s, axis=1, keepdims=True)
    hi = rowmax
    for _ in range(ITERS):
        mid = 0.5 * (lo + hi)
        cnt = jnp.sum(jnp.where(s >= mid, 1.0, 0.0), axis=1, keepdims=True)
        ge = cnt >= KTOP
        lo = jnp.where(ge, mid, lo)
        hi = jnp.where(ge, hi, mid)
    e = jnp.where(s >= lo, jnp.exp(s - rowmax), 0.0)
    z = jnp.sum(e, axis=1, keepdims=True)
    ev = jax.lax.dot_general(e, v_ref[...], (((1,), (0,)), ((), ())),
                             preferred_element_type=jnp.float32)
    o_ref[...] = ev / z


def _mid_kernel(a_ref, x_ref, pw_ref, g_ref, b_ref, fw_ref, x1_ref, h0_ref):
    a = jnp.concatenate([a_ref[h2, t] for t in range(T) for h2 in range(HEADS)],
                        axis=1)
    y = jax.lax.dot_general(a, pw_ref[...], (((1,), (1,)), ((), ())),
                            preferred_element_type=jnp.float32)
    x1 = x_ref[...].T + y
    x1_ref[...] = x1
    xn = _ln_tm(x1, g_ref[...], b_ref[...])
    h0_ref[...] = jax.lax.dot_general(xn, fw_ref[...], (((1,), (1,)), ((), ())),
                                      preferred_element_type=jnp.float32)


def _ffn_kernel(xp_ref, w_ref, ow_ref, x1_ref, o_ref):
    acc = _conv9(xp_ref, w_ref)
    hact = jnp.maximum(acc, 0.0).reshape(POS, HF)
    y = jax.lax.dot_general(hact, ow_ref[...], (((1,), (1,)), ((), ())),
                            preferred_element_type=jnp.float32)
    o_ref[...] = (x1_ref[...] + y).T


def kernel(x, norm1_w, norm1_b, qkv_w, qkv_dw_w, temperature, proj_w,
           norm2_w, norm2_b, ffn_in_w, ffn_dw_w, ffn_out_w):
    f32 = jnp.float32
    x_cm = x.reshape(B, DIM, POS)

    wmix = jnp.einsum('ot,cd->tcod', qkv_w, jnp.eye(CG, dtype=f32)).reshape(DIM, 3 * T * CG)

    y = pl.pallas_call(
        _ln_qkv_kernel,
        grid=(B,),
        out_shape=jax.ShapeDtypeStruct((B, POS, 3 * T * CG), f32),
        in_specs=[pl.BlockSpec((None, DIM, POS), lambda i: (i, 0, 0)),
                  pl.BlockSpec((DIM, 1), lambda i: (0, 0)),
                  pl.BlockSpec((DIM, 1), lambda i: (0, 0)),
                  pl.BlockSpec((DIM, 3 * T * CG), lambda i: (0, 0))],
        out_specs=pl.BlockSpec((None, POS, 3 * T * CG), lambda i: (i, 0, 0)),
    )(x_cm, norm1_w.reshape(DIM, 1), norm1_b.reshape(DIM, 1), wmix)

    ypad = jnp.pad(y.reshape(B, H, W, 3 * T * CG), ((0, 0), (1, 1), (1, 1), (0, 0)))
    wtap_q = jnp.repeat(qkv_dw_w[:, 0].transpose(1, 2, 0).reshape(9, 3 * T), CG, axis=1)
    tvec = temperature.reshape(HEADS, 1, 1)

    qn, kn, v8 = pl.pallas_call(
        _dwconv_kernel,
        grid=(B,),
        out_shape=[jax.ShapeDtypeStruct((B, HEADS, N, CPH), f32),
                   jax.ShapeDtypeStruct((B, CG, N), f32),
                   jax.ShapeDtypeStruct((B, HEADS, N, CPH), f32)],
        in_specs=[pl.BlockSpec((None, H + 2, W + 2, 3 * T * CG), lambda i: (i, 0, 0, 0)),
                  pl.BlockSpec((9, 3 * T * CG), lambda i: (0, 0)),
                  pl.BlockSpec((HEADS, 1, 1), lambda i: (0, 0, 0))],
        out_specs=[pl.BlockSpec((None, HEADS, N, CPH), lambda i: (i, 0, 0, 0)),
                   pl.BlockSpec((None, CG, N), lambda i: (i, 0, 0)),
                   pl.BlockSpec((None, HEADS, N, CPH), lambda i: (i, 0, 0, 0))],
    )(ypad, wtap_q, tvec)

    qn = qn.reshape(BH, N, CPH)
    kn = kn.reshape(BH, CPH, N)
    v8 = v8.reshape(BH, N, CPH)

    outt = pl.pallas_call(
        _attn_kernel,
        grid=(BH, N // ROWS),
        out_shape=jax.ShapeDtypeStruct((BH, N, CPH), f32),
        in_specs=[pl.BlockSpec((None, ROWS, CPH), lambda i, j: (i, j, 0)),
                  pl.BlockSpec((None, CPH, N), lambda i, j: (i, 0, 0)),
                  pl.BlockSpec((None, N, CPH), lambda i, j: (i, 0, 0))],
        out_specs=pl.BlockSpec((None, ROWS, CPH), lambda i, j: (i, j, 0)),
        compiler_params=pltpu.CompilerParams(
            dimension_semantics=("arbitrary", "arbitrary")),
    )(qn, kn, v8)

    a5 = outt.reshape(B, HEADS, T, POS, CPH)
    x1, h0 = pl.pallas_call(
        _mid_kernel,
        grid=(B,),
        out_shape=[jax.ShapeDtypeStruct((B, POS, DIM), f32),
                   jax.ShapeDtypeStruct((B, POS, HF), f32)],
        in_specs=[pl.BlockSpec((None, HEADS, T, POS, CPH), lambda i: (i, 0, 0, 0, 0)),
                  pl.BlockSpec((None, DIM, POS), lambda i: (i, 0, 0)),
                  pl.BlockSpec((DIM, DIM), lambda i: (0, 0)),
                  pl.BlockSpec((1, DIM), lambda i: (0, 0)),
                  pl.BlockSpec((1, DIM), lambda i: (0, 0)),
                  pl.BlockSpec((HF, DIM), lambda i: (0, 0))],
        out_specs=[pl.BlockSpec((None, POS, DIM), lambda i: (i, 0, 0)),
                   pl.BlockSpec((None, POS, HF), lambda i: (i, 0, 0))],
    )(a5, x_cm, proj_w, norm2_w.reshape(1, DIM), norm2_b.reshape(1, DIM), ffn_in_w)

    h0pad = jnp.pad(h0.reshape(B, H, W, HF), ((0, 0), (1, 1), (1, 1), (0, 0)))
    wtap_f = ffn_dw_w[:, 0].transpose(1, 2, 0).reshape(9, HF)

    out = pl.pallas_call(
        _ffn_kernel,
        grid=(B,),
        out_shape=jax.ShapeDtypeStruct((B, DIM, POS), f32),
        in_specs=[pl.BlockSpec((None, H + 2, W + 2, HF), lambda i: (i, 0, 0, 0)),
                  pl.BlockSpec((9, HF), lambda i: (0, 0)),
                  pl.BlockSpec((DIM, HF), lambda i: (0, 0)),
                  pl.BlockSpec((None, POS, DIM), lambda i: (i, 0, 0))],
        out_specs=pl.BlockSpec((None, DIM, POS), lambda i: (i, 0, 0)),
    )(h0pad, wtap_f, ffn_out_w, x1)

    return out.reshape(B, DIM, H, W)


# speedup vs baseline: 152.9393x; 1.0362x over previous
import jax
import jax.numpy as jnp
from jax.experimental import pallas as pl
from jax.experimental.pallas import tpu as pltpu

DIM = 128
HEADS = 4
T = 2
HF = 64
H = W = 32
B = 2
POS = H * W
N = POS * T
CG = DIM // T
CPH = CG // HEADS
BH = B * HEADS
KTOP = int(N * 0.25)
ROWS = 512
ITERS = 12


def _ln_cm(x, g, b):
    mu = jnp.mean(x, axis=0, keepdims=True)
    xc = x - mu
    var = jnp.mean(xc * xc, axis=0, keepdims=True)
    return xc * jax.lax.rsqrt(var + 1e-5) * g + b


def _ln_tm(x, g, b):
    mu = jnp.mean(x, axis=1, keepdims=True)
    xc = x - mu
    var = jnp.mean(xc * xc, axis=1, keepdims=True)
    return xc * jax.lax.rsqrt(var + 1e-5) * g + b


def _pad_conv9(y, w_ref, ch):
    yim = y.reshape(H, W, ch)
    zc = jnp.zeros((H, 1, ch), jnp.float32)
    yim = jnp.concatenate([zc, yim, zc], axis=1)
    zr = jnp.zeros((1, W + 2, ch), jnp.float32)
    yim = jnp.concatenate([zr, yim, zr], axis=0)
    acc = yim[0:H, 0:W, :] * w_ref[0:1, :].reshape(1, 1, ch)
    for tap in range(1, 9):
        dy, dx = tap // 3, tap % 3
        acc = acc + yim[dy:dy + H, dx:dx + W, :] * w_ref[tap:tap + 1, :].reshape(1, 1, ch)
    return acc


def _pre_kernel(x_ref, g_ref, b_ref, wmix_ref, w_ref, t_ref, q_ref, k_ref, v_ref):
    xn = _ln_cm(x_ref[...], g_ref[...], b_ref[...])
    y = jax.lax.dot_general(xn, wmix_ref[...], (((0,), (0,)), ((), ())),
                            preferred_element_type=jnp.float32)
    acc = _pad_conv9(y, w_ref, 3 * T * CG).reshape(POS, 3 * T * CG)
    qs = [jnp.concatenate([acc[:, t * CG + h2 * CPH:t * CG + (h2 + 1) * CPH]
                           for t in range(T)], axis=0) for h2 in range(HEADS)]
    vs = [jnp.concatenate([acc[:, (2 * T + t) * CG + h2 * CPH:(2 * T + t) * CG + (h2 + 1) * CPH]
                           for t in range(T)], axis=0) for h2 in range(HEADS)]
    q = jnp.stack(qs)
    v = jnp.stack(vs)
    k = jnp.concatenate([acc[:, (T + t) * CG:(T + 1 + t) * CG].T
                         for t in range(T)], axis=1)
    qn = jnp.maximum(jnp.sqrt(jnp.sum(q * q, axis=1, keepdims=True)), 1e-12)
    kn = jnp.maximum(jnp.sqrt(jnp.sum(k * k, axis=1, keepdims=True)), 1e-12)
    q_ref[...] = (q / qn) * t_ref[...]
    k_ref[...] = k / kn
    v_ref[...] = v


def _attn_kernel(q_ref, k_ref, v_ref, o_ref):
    s = jax.lax.dot_general(q_ref[...], k_ref[...], (((1,), (0,)), ((), ())),
                            preferred_element_type=jnp.float32)
    rowmax = jnp.max(s, axis=1, keepdims=True)
    lo = jnp.min(s, axis=1, keepdims=True)
    hi = rowmax
    for _ in range(ITERS):
        mid = 0.5 * (lo + hi)
        cnt = jnp.sum(jnp.where(s >= mid, 1.0, 0.0), axis=1, keepdims=True)
        ge = cnt >= KTOP
        lo = jnp.where(ge, mid, lo)
        hi = jnp.where(ge, hi, mid)
    e = jnp.where(s >= lo, jnp.exp(s - rowmax), 0.0)
    z = jnp.sum(e, axis=1, keepdims=True)
    ev = jax.lax.dot_general(e, v_ref[...], (((1,), (0,)), ((), ())),
                             preferred_element_type=jnp.float32)
    o_ref[...] = ev / z


def _post_kernel(a_ref, x_ref, pw_ref, g_ref, b_ref, fw_ref, wf_ref, ow_ref, o_ref):
    a = jnp.concatenate([a_ref[h2, t] for t in range(T) for h2 in range(HEADS)],
                        axis=1)
    y = jax.lax.dot_general(a, pw_ref[...], (((1,), (1,)), ((), ())),
                            preferred_element_type=jnp.float32)
    x1 = x_ref[...].T + y
    xn = _ln_tm(x1, g_ref[...], b_ref[...])
    h0 = jax.lax.dot_general(xn, fw_ref[...], (((1,), (1,)), ((), ())),
                             preferred_element_type=jnp.float32)
    hact = jnp.maximum(_pad_conv9(h0, wf_ref, HF), 0.0).reshape(POS, HF)
    y2 = jax.lax.dot_general(hact, ow_ref[...], (((1,), (1,)), ((), ())),
                             preferred_element_type=jnp.float32)
    o_ref[...] = (x1 + y2).T


def kernel(x, norm1_w, norm1_b, qkv_w, qkv_dw_w, temperature, proj_w,
           norm2_w, norm2_b, ffn_in_w, ffn_dw_w, ffn_out_w):
    f32 = jnp.float32
    x_cm = x.reshape(B, DIM, POS)

    wmix = jnp.einsum('ot,cd->tcod', qkv_w, jnp.eye(CG, dtype=f32)).reshape(DIM, 3 * T * CG)
    wtap_q = jnp.repeat(qkv_dw_w[:, 0].transpose(1, 2, 0).reshape(9, 3 * T), CG, axis=1)
    wtap_f = ffn_dw_w[:, 0].transpose(1, 2, 0).reshape(9, HF)
    tvec = temperature.reshape(HEADS, 1, 1)

    qn, kn, v8 = pl.pallas_call(
        _pre_kernel,
        grid=(B,),
        out_shape=[jax.ShapeDtypeStruct((B, HEADS, N, CPH), f32),
                   jax.ShapeDtypeStruct((B, CG, N), f32),
                   jax.ShapeDtypeStruct((B, HEADS, N, CPH), f32)],
        in_specs=[pl.BlockSpec((None, DIM, POS), lambda i: (i, 0, 0)),
                  pl.BlockSpec((DIM, 1), lambda i: (0, 0)),
                  pl.BlockSpec((DIM, 1), lambda i: (0, 0)),
                  pl.BlockSpec((DIM, 3 * T * CG), lambda i: (0, 0)),
                  pl.BlockSpec((9, 3 * T * CG), lambda i: (0, 0)),
                  pl.BlockSpec((HEADS, 1, 1), lambda i: (0, 0, 0))],
        out_specs=[pl.BlockSpec((None, HEADS, N, CPH), lambda i: (i, 0, 0, 0)),
                   pl.BlockSpec((None, CG, N), lambda i: (i, 0, 0)),
                   pl.BlockSpec((None, HEADS, N, CPH), lambda i: (i, 0, 0, 0))],
    )(x_cm, norm1_w.reshape(DIM, 1), norm1_b.reshape(DIM, 1), wmix, wtap_q, tvec)

    qn = qn.reshape(BH, N, CPH)
    kn = kn.reshape(BH, CPH, N)
    v8 = v8.reshape(BH, N, CPH)

    outt = pl.pallas_call(
        _attn_kernel,
        grid=(BH, N // ROWS),
        out_shape=jax.ShapeDtypeStruct((BH, N, CPH), f32),
        in_specs=[pl.BlockSpec((None, ROWS, CPH), lambda i, j: (i, j, 0)),
                  pl.BlockSpec((None, CPH, N), lambda i, j: (i, 0, 0)),
                  pl.BlockSpec((None, N, CPH), lambda i, j: (i, 0, 0))],
        out_specs=pl.BlockSpec((None, ROWS, CPH), lambda i, j: (i, j, 0)),
        compiler_params=pltpu.CompilerParams(
            dimension_semantics=("arbitrary", "arbitrary")),
    )(qn, kn, v8)

    a5 = outt.reshape(B, HEADS, T, POS, CPH)
    out = pl.pallas_call(
        _post_kernel,
        grid=(B,),
        out_shape=jax.ShapeDtypeStruct((B, DIM, POS), f32),
        in_specs=[pl.BlockSpec((None, HEADS, T, POS, CPH), lambda i: (i, 0, 0, 0, 0)),
                  pl.BlockSpec((None, DIM, POS), lambda i: (i, 0, 0)),
                  pl.BlockSpec((DIM, DIM), lambda i: (0, 0)),
                  pl.BlockSpec((1, DIM), lambda i: (0, 0)),
                  pl.BlockSpec((1, DIM), lambda i: (0, 0)),
                  pl.BlockSpec((HF, DIM), lambda i: (0, 0)),
                  pl.BlockSpec((9, HF), lambda i: (0, 0)),
                  pl.BlockSpec((DIM, HF), lambda i: (0, 0))],
        out_specs=pl.BlockSpec((None, DIM, POS), lambda i: (i, 0, 0)),
    )(a5, x_cm, proj_w, norm2_w.reshape(1, DIM), norm2_b.reshape(1, DIM),
      ffn_in_w, wtap_f, ffn_out_w)

    return out.reshape(B, DIM, H, W)
